# baseline (speedup 1.0000x reference)
import functools

import jax
import jax.numpy as jnp
from jax.experimental import pallas as pl
from jax.experimental.pallas import tpu as pltpu

_EPS = 1e-5


def _stats_kernel(n_rows, half_m, needs_mask, ma_ref, mb_ref, out_ref):
    s = jnp.zeros((1, ma_ref.shape[1]), jnp.float32)
    ss = s
    for k, ref in enumerate((ma_ref, mb_ref)):
        m = ref[...].astype(jnp.float32)
        if needs_mask:
            base = (pl.program_id(0) * 2 + k) * half_m
            rid = jax.lax.broadcasted_iota(jnp.int32, m.shape, 0) + base
            m = jnp.where(rid < n_rows, m, 0.0)
        s = s + jnp.sum(m, axis=0, keepdims=True)
        ss = ss + jnp.sum(m * m, axis=0, keepdims=True)
    row = jax.lax.broadcasted_iota(jnp.int32, (8, s.shape[1]), 0)
    out_ref[...] = jnp.where(row == 0, s, jnp.where(row == 1, ss, 0.0))


def _encoder_core(dim_x, n_rows, part_ref, meas_ref, x_ref, gamma_ref,
                  beta_ref, w1_ref, b1_ref, w2_ref, b2_ref, wx_ref, bx_ref):
    f32 = jnp.float32
    p = part_ref[...]
    prow = jax.lax.broadcasted_iota(jnp.int32, p.shape, 0)
    s = jnp.sum(jnp.where(prow % 8 == 0, p, 0.0), axis=0, keepdims=True)
    ss = jnp.sum(jnp.where(prow % 8 == 1, p, 0.0), axis=0, keepdims=True)
    inv_n = 1.0 / n_rows
    mean = s * inv_n
    var = jnp.maximum(ss * inv_n - mean * mean, 0.0)
    scale = gamma_ref[...].astype(f32) * jax.lax.rsqrt(var + _EPS)
    shift = beta_ref[...].astype(f32) - mean * scale

    w1 = w1_ref[...].astype(f32)
    w1f = w1 * scale.reshape(-1, 1)
    b1f = jnp.dot(shift, w1, preferred_element_type=f32) + b1_ref[...].astype(f32)

    h1 = jnp.dot(meas_ref[...].astype(f32), w1f, preferred_element_type=f32)
    h1 = jnp.maximum(h1 + b1f, 0.0)

    two_pe = h1.shape[1]
    wx = wx_ref[...].astype(f32).T
    w2 = w2_ref[...].astype(f32).T
    wtop = jnp.concatenate(
        [wx, jnp.zeros((wx.shape[0], two_pe // 2), f32)], axis=1)
    wbot = jnp.concatenate(
        [jnp.zeros((two_pe, dim_x), f32), w2], axis=1)
    wcat = jnp.concatenate([wtop, wbot], axis=0)
    bcat = jnp.concatenate([bx_ref[...].astype(f32),
                            b2_ref[...].astype(f32)], axis=1)

    xh = jnp.concatenate([x_ref[...].astype(f32), h1], axis=1)
    z = jnp.dot(xh, wcat, preferred_element_type=f32) + bcat

    col = jax.lax.broadcasted_iota(jnp.int32, z.shape, 1)
    return jnp.where(col >= dim_x, jnp.maximum(z, 0.0), z)


def _encoder_kernel(dim_x, n_rows, *refs):
    out_ref, pet_ref = refs[-2:]
    z = _encoder_core(dim_x, n_rows, *refs[:-2])
    out_ref[...] = z.astype(out_ref.dtype)
    pet_ref[...] = z[:, dim_x:].T.astype(pet_ref.dtype)


def _encoder_kernel_single(dim_x, n_rows, *refs):
    out_ref = refs[-1]
    z = _encoder_core(dim_x, n_rows, *refs[:-1])
    out_ref[...] = z.astype(out_ref.dtype)


def kernel(x, measure, gamma, beta, w1, b1, w2, b2, wx, bx):
    dim_emb, dim_pe = 256, 64
    N, dim_in = x.shape
    _, num_metrics = measure.shape
    dim_x = dim_emb - dim_pe

    stats_tile = min(16384, ((N + 15) // 16) * 16)
    half_m = stats_tile // 2
    n_stat_tiles = pl.cdiv(N, stats_tile)
    needs_mask = (N % stats_tile) != 0
    partial = pl.pallas_call(
        functools.partial(_stats_kernel, N, half_m, needs_mask),
        out_shape=jax.ShapeDtypeStruct((n_stat_tiles * 8, num_metrics),
                                       jnp.float32),
        grid=(n_stat_tiles,),
        in_specs=[
            pl.BlockSpec((half_m, num_metrics), lambda i: (2 * i, 0)),
            pl.BlockSpec((half_m, num_metrics), lambda i: (2 * i + 1, 0)),
        ],
        out_specs=pl.BlockSpec((8, num_metrics), lambda i: (i, 0)),
        compiler_params=pltpu.CompilerParams(
            dimension_semantics=("parallel",),
            vmem_limit_bytes=64 * 1024 * 1024,
        ),
    )(measure, measure)

    tile_m = min(4096, ((N + 7) // 8) * 8)
    grid = (pl.cdiv(N, tile_m),)

    def full_spec(arr):
        return pl.BlockSpec(arr.shape, lambda i: (0, 0))

    w2t = w2.T
    wxt = wx.T

    out, pet = pl.pallas_call(
        functools.partial(_encoder_kernel, dim_x, N),
        out_shape=(jax.ShapeDtypeStruct((N, dim_emb), x.dtype),
                   jax.ShapeDtypeStruct((dim_pe, N), jnp.float32)),
        grid=grid,
        in_specs=[
            full_spec(partial),
            pl.BlockSpec((tile_m, num_metrics), lambda i: (i, 0)),
            pl.BlockSpec((tile_m, dim_in), lambda i: (i, 0)),
            full_spec(gamma), full_spec(beta), full_spec(w1), full_spec(b1),
            full_spec(w2t), full_spec(b2), full_spec(wxt), full_spec(bx),
        ],
        out_specs=(pl.BlockSpec((tile_m, dim_emb), lambda i: (i, 0)),
                   pl.BlockSpec((dim_pe, tile_m), lambda i: (0, i))),
        compiler_params=pltpu.CompilerParams(
            dimension_semantics=("parallel",),
            vmem_limit_bytes=64 * 1024 * 1024,
        ),
    )(partial, measure, x, gamma, beta, w1, b1, w2t, b2, wxt, bx)
    pe = pet.T
    return out, pe


# speedup vs baseline: 1.0315x; 1.0315x over previous
import functools

import jax
import jax.numpy as jnp
from jax.experimental import pallas as pl
from jax.experimental.pallas import tpu as pltpu

_EPS = 1e-5


def _stats_kernel(n_rows, half_m, needs_mask, ma_ref, mb_ref, out_ref):
    s = jnp.zeros((1, ma_ref.shape[1]), jnp.float32)
    ss = s
    for k, ref in enumerate((ma_ref, mb_ref)):
        m = ref[...].astype(jnp.float32)
        if needs_mask:
            base = (pl.program_id(0) * 2 + k) * half_m
            rid = jax.lax.broadcasted_iota(jnp.int32, m.shape, 0) + base
            m = jnp.where(rid < n_rows, m, 0.0)
        s = s + jnp.sum(m, axis=0, keepdims=True)
        ss = ss + jnp.sum(m * m, axis=0, keepdims=True)
    row = jax.lax.broadcasted_iota(jnp.int32, (8, s.shape[1]), 0)
    out_ref[...] = jnp.where(row == 0, s, jnp.where(row == 1, ss, 0.0))


def _encoder_core(dim_x, n_rows, part_ref, meas_ref, x_ref, gamma_ref,
                  beta_ref, w1_ref, b1_ref, w2_ref, b2_ref, wx_ref, bx_ref):
    f32 = jnp.float32
    p = part_ref[...]
    prow = jax.lax.broadcasted_iota(jnp.int32, p.shape, 0)
    s = jnp.sum(jnp.where(prow % 8 == 0, p, 0.0), axis=0, keepdims=True)
    ss = jnp.sum(jnp.where(prow % 8 == 1, p, 0.0), axis=0, keepdims=True)
    inv_n = 1.0 / n_rows
    mean = s * inv_n
    var = jnp.maximum(ss * inv_n - mean * mean, 0.0)
    scale = gamma_ref[...].astype(f32) * jax.lax.rsqrt(var + _EPS)
    shift = beta_ref[...].astype(f32) - mean * scale

    w1 = w1_ref[...].astype(f32)
    w1f = w1 * scale.reshape(-1, 1)
    b1f = jnp.dot(shift, w1, preferred_element_type=f32) + b1_ref[...].astype(f32)

    h1 = jnp.dot(meas_ref[...].astype(f32), w1f, preferred_element_type=f32)
    h1 = jnp.maximum(h1 + b1f, 0.0)

    two_pe = h1.shape[1]
    wx = wx_ref[...].astype(f32).T
    w2 = w2_ref[...].astype(f32).T
    wtop = jnp.concatenate(
        [wx, jnp.zeros((wx.shape[0], two_pe // 2), f32)], axis=1)
    wbot = jnp.concatenate(
        [jnp.zeros((two_pe, dim_x), f32), w2], axis=1)
    wcat = jnp.concatenate([wtop, wbot], axis=0)
    bcat = jnp.concatenate([bx_ref[...].astype(f32),
                            b2_ref[...].astype(f32)], axis=1)

    xh = jnp.concatenate([x_ref[...].astype(f32), h1], axis=1)
    z = jnp.dot(xh, wcat, preferred_element_type=f32) + bcat

    col = jax.lax.broadcasted_iota(jnp.int32, z.shape, 1)
    return jnp.where(col >= dim_x, jnp.maximum(z, 0.0), z)


def _encoder_kernel(dim_x, n_rows, part_ref, ma_ref, mb_ref, xa_ref, xb_ref,
                    gamma_ref, beta_ref, w1_ref, b1_ref, w2_ref, b2_ref,
                    wx_ref, bx_ref, out_ref, pet_ref):
    half = ma_ref.shape[0]
    for k, (mref, xref) in enumerate(((ma_ref, xa_ref), (mb_ref, xb_ref))):
        z = _encoder_core(dim_x, n_rows, part_ref, mref, xref, gamma_ref,
                          beta_ref, w1_ref, b1_ref, w2_ref, b2_ref, wx_ref,
                          bx_ref)
        rows = pl.ds(k * half, half)
        out_ref[rows, :] = z.astype(out_ref.dtype)
        pet_ref[:, rows] = z[:, dim_x:].T.astype(pet_ref.dtype)


def kernel(x, measure, gamma, beta, w1, b1, w2, b2, wx, bx):
    dim_emb, dim_pe = 256, 64
    N, dim_in = x.shape
    _, num_metrics = measure.shape
    dim_x = dim_emb - dim_pe

    stats_tile = min(16384, ((N + 15) // 16) * 16)
    half_m = stats_tile // 2
    n_stat_tiles = pl.cdiv(N, stats_tile)
    needs_mask = (N % stats_tile) != 0
    partial = pl.pallas_call(
        functools.partial(_stats_kernel, N, half_m, needs_mask),
        out_shape=jax.ShapeDtypeStruct((n_stat_tiles * 8, num_metrics),
                                       jnp.float32),
        grid=(n_stat_tiles,),
        in_specs=[
            pl.BlockSpec((half_m, num_metrics), lambda i: (2 * i, 0)),
            pl.BlockSpec((half_m, num_metrics), lambda i: (2 * i + 1, 0)),
        ],
        out_specs=pl.BlockSpec((8, num_metrics), lambda i: (i, 0)),
        compiler_params=pltpu.CompilerParams(
            dimension_semantics=("parallel",),
            vmem_limit_bytes=64 * 1024 * 1024,
        ),
    )(measure, measure)

    tile_m = min(8192, ((N + 15) // 16) * 16)
    half_t = tile_m // 2
    grid = (pl.cdiv(N, tile_m),)

    def full_spec(arr):
        return pl.BlockSpec(arr.shape, lambda i: (0, 0))

    w2t = w2.T
    wxt = wx.T

    out, pet = pl.pallas_call(
        functools.partial(_encoder_kernel, dim_x, N),
        out_shape=(jax.ShapeDtypeStruct((N, dim_emb), x.dtype),
                   jax.ShapeDtypeStruct((dim_pe, N), jnp.float32)),
        grid=grid,
        in_specs=[
            full_spec(partial),
            pl.BlockSpec((half_t, num_metrics), lambda i: (2 * i, 0)),
            pl.BlockSpec((half_t, num_metrics), lambda i: (2 * i + 1, 0)),
            pl.BlockSpec((half_t, dim_in), lambda i: (2 * i, 0)),
            pl.BlockSpec((half_t, dim_in), lambda i: (2 * i + 1, 0)),
            full_spec(gamma), full_spec(beta), full_spec(w1), full_spec(b1),
            full_spec(w2t), full_spec(b2), full_spec(wxt), full_spec(bx),
        ],
        out_specs=(pl.BlockSpec((tile_m, dim_emb), lambda i: (i, 0)),
                   pl.BlockSpec((dim_pe, tile_m), lambda i: (0, i))),
        compiler_params=pltpu.CompilerParams(
            dimension_semantics=("parallel",),
            vmem_limit_bytes=64 * 1024 * 1024,
        ),
    )(partial, measure, measure, x, x, gamma, beta, w1, b1, w2t, b2, wxt, bx)
    pe = pet.T
    return out, pe
